# baseline (speedup 1.0000x reference)
import jax
import jax.numpy as jnp
from jax import lax
from jax.experimental import pallas as pl
from jax.experimental.pallas import tpu as pltpu
from jax.experimental.pallas import tpu_sc as plsc

N_NODES = 10000
D = 128
W_AUG = 144

NC = 2
NS = 16
NW = NC * NS

CHUNK = 128
K0 = 160
K1 = 0
E_PAD = NS * (K0 + K1) * CHUNK
N_ACC = 10240
ROWS_PER_TILE = N_ACC // NS


def _sc_body(feat_hbm, src_hbm, dst_hbm, out_hbm,
             rows0, rows1, sidx0, sidx1, didx0, didx1, acc,
             gsem0, gsem1, ssem0, ssem1, sisem0, sisem1, disem0, disem1):
    c = lax.axis_index("c")
    s = lax.axis_index("s")
    wid = s * NC + c

    rows = (rows0, rows1)
    sidx = (sidx0, sidx1)
    didx = (didx0, didx1)
    gsem = (gsem0, gsem1)
    ssem = (ssem0, ssem1)
    sisem = (sisem0, sisem1)
    disem = (disem0, disem1)

    zero16 = jnp.zeros((16,), jnp.float32)

    def zrow(i, _):
        def zcol(j, _):
            rows0[i, pl.ds(j * 16, 16)] = zero16
            return ()
        return lax.fori_loop(0, W_AUG // 16, zcol, ())

    lax.fori_loop(0, CHUNK, zrow, ())

    row0 = s * ROWS_PER_TILE
    for k in range(ROWS_PER_TILE // CHUNK):
        pltpu.sync_copy(rows0, acc.at[pl.ds(row0 + k * CHUNK, CHUNK)])
    rem = ROWS_PER_TILE % CHUNK
    if rem:
        base = row0 + (ROWS_PER_TILE // CHUNK) * CHUNK
        pltpu.sync_copy(rows0.at[pl.ds(0, rem)], acc.at[pl.ds(base, rem)])

    plsc.subcore_barrier()

    nk = jnp.where(c == 0, K0, K1)
    cbase = jnp.where(c == 0, s * K0, NS * K0 + s * K1)
    ebase = cbase * CHUNK

    def sidx_load(b, g):
        pltpu.async_copy(src_hbm.at[pl.ds(ebase + g * CHUNK, CHUNK)],
                         sidx[b], sisem[b])

    def sidx_wait(b):
        pltpu.make_async_copy(src_hbm.at[pl.ds(0, CHUNK)], sidx[b],
                              sisem[b]).wait()

    def didx_load(b, g):
        pltpu.async_copy(dst_hbm.at[pl.ds(ebase + g * CHUNK, CHUNK)],
                         didx[b], disem[b])

    def didx_wait(b):
        pltpu.make_async_copy(dst_hbm.at[pl.ds(0, CHUNK)], didx[b],
                              disem[b]).wait()

    def gather_start(b):
        pltpu.async_copy(feat_hbm.at[sidx[b]], rows[b], gsem[b])

    def gather_wait(b):
        pltpu.make_async_copy(feat_hbm.at[sidx[b]], rows[b], gsem[b]).wait()

    def scatter_start(b):
        pltpu.async_copy(rows[b], acc.at[didx[b]], ssem[b], add=True)

    def scatter_wait(b):
        pltpu.make_async_copy(rows[b], acc.at[didx[b]], ssem[b]).wait()

    def body(b, g, first=False, no_next=False, no_sidx=False):
        ob = 1 - b
        gather_wait(b)
        if not first:
            scatter_wait(ob)
        if not no_next:
            sidx_wait(ob)
            gather_start(ob)
            if not no_sidx:
                sidx_load(b, g + 2)
            didx_load(ob, g + 1)
        didx_wait(b)
        scatter_start(b)

    @pl.when(c == 0)
    def _main():
        sidx_load(0, 0)
        didx_load(0, 0)
        sidx_load(1, 1)
        sidx_wait(0)
        gather_start(0)

        body(0, 0, first=True)
        body(1, 1)

        def pair(p, _):
            g = 2 * p
            body(0, g)
            body(1, g + 1)
            return ()

        lax.fori_loop(1, (nk - 2) // 2, pair, ())

        body(0, nk - 2, no_sidx=True)
        body(1, nk - 1, no_next=True)
        scatter_wait(1)

    plsc.subcore_barrier()

    pltpu.sync_copy(acc.at[pl.ds(row0, ROWS_PER_TILE)],
                    out_hbm.at[c, pl.ds(row0, ROWS_PER_TILE)])


@jax.jit
def _sc_partials(feat_aug, src_p, dst_p):
    mesh = plsc.VectorSubcoreMesh(core_axis_name="c", subcore_axis_name="s")
    return pl.kernel(
        _sc_body,
        out_type=jax.ShapeDtypeStruct((NC, N_ACC, W_AUG), jnp.float32),
        mesh=mesh,
        scratch_types=[
            pltpu.VMEM((CHUNK, W_AUG), jnp.float32),
            pltpu.VMEM((CHUNK, W_AUG), jnp.float32),
            pltpu.VMEM((CHUNK,), jnp.int32),
            pltpu.VMEM((CHUNK,), jnp.int32),
            pltpu.VMEM((CHUNK,), jnp.int32),
            pltpu.VMEM((CHUNK,), jnp.int32),
            pltpu.VMEM_SHARED((N_ACC, W_AUG), jnp.float32),
        ] + [pltpu.SemaphoreType.DMA] * 8,
        compiler_params=pltpu.CompilerParams(use_tc_tiling_on_sc=False),
    )(feat_aug, src_p, dst_p)


RB = 400


def _tc_body(p_ref, wgc_ref, bgc_ref, wlt_ref, bl_ref, o_ref):
    x = p_ref[...]
    st = x[0] + x[1]
    deg = jnp.clip(st[:, D], 1.0, None)
    h = st[:, :D] / deg[:, None]
    ge = jnp.dot(h, wgc_ref[...], preferred_element_type=jnp.float32)
    ge = jnp.maximum(ge + bgc_ref[...], 0.0)
    top = jnp.max(ge, axis=1, keepdims=True)
    cnt = jnp.sum((ge == top).astype(jnp.float32), axis=1, keepdims=True)
    h2 = h * cnt
    o_ref[...] = (jnp.dot(h2, wlt_ref[...], preferred_element_type=jnp.float32)
                  + bl_ref[...])


@jax.jit
def _tc_finish(parts, W_gc, b_gc, W_lin_t, b_lin2d):
    grid = N_NODES // RB
    return pl.pallas_call(
        _tc_body,
        grid=(grid,),
        in_specs=[
            pl.BlockSpec((NC, RB, W_AUG), lambda i: (0, i, 0)),
            pl.BlockSpec((D, 3), lambda i: (0, 0)),
            pl.BlockSpec((1, 3), lambda i: (0, 0)),
            pl.BlockSpec((D, D), lambda i: (0, 0)),
            pl.BlockSpec((1, D), lambda i: (0, 0)),
        ],
        out_specs=pl.BlockSpec((RB, D), lambda i: (i, 0)),
        out_shape=jax.ShapeDtypeStruct((N_NODES, D), jnp.float32),
    )(parts, W_gc, b_gc, W_lin_t, b_lin2d)


def kernel(feature, edge_index, W_gc, b_gc, W_lin, b_lin):
    src = edge_index[0].astype(jnp.int32)
    dst = edge_index[1].astype(jnp.int32)
    e = src.shape[0]
    pad = E_PAD - e
    src_p = jnp.concatenate([src, jnp.zeros((pad,), jnp.int32)])
    pad_dst = N_NODES + (jnp.arange(pad, dtype=jnp.int32) % (N_ACC - N_NODES))
    dst_p = jnp.concatenate([dst, pad_dst])
    feat_aug = jnp.concatenate(
        [feature,
         jnp.ones((N_NODES, 1), jnp.float32),
         jnp.zeros((N_NODES, W_AUG - D - 1), jnp.float32)], axis=1)
    parts = _sc_partials(feat_aug, src_p, dst_p)
    return _tc_finish(parts, W_gc, b_gc, W_lin.T, b_lin.reshape(1, D))


# speedup vs baseline: 2.8600x; 2.8600x over previous
import jax
import jax.numpy as jnp
from jax import lax
from jax.experimental import pallas as pl
from jax.experimental.pallas import tpu as pltpu
from jax.experimental.pallas import tpu_sc as plsc

N_NODES = 10000
D = 128
W_AUG = 144

NC = 2
NS = 16
NW = NC * NS

CHUNK = 128
K0 = 80
K1 = 80
E_PAD = NS * (K0 + K1) * CHUNK
N_ACC = 10240
ROWS_PER_TILE = N_ACC // NS


def _sc_body(feat_hbm, src_hbm, dst_hbm, out_hbm,
             rows0, rows1, sidx0, sidx1, didx0, didx1, acc,
             gsem0, gsem1, ssem0, ssem1, sisem0, sisem1, disem0, disem1):
    c = lax.axis_index("c")
    s = lax.axis_index("s")
    wid = s * NC + c

    rows = (rows0, rows1)
    sidx = (sidx0, sidx1)
    didx = (didx0, didx1)
    gsem = (gsem0, gsem1)
    ssem = (ssem0, ssem1)
    sisem = (sisem0, sisem1)
    disem = (disem0, disem1)

    zero16 = jnp.zeros((16,), jnp.float32)

    def zrow(i, _):
        def zcol(j, _):
            rows0[i, pl.ds(j * 16, 16)] = zero16
            return ()
        return lax.fori_loop(0, W_AUG // 16, zcol, ())

    lax.fori_loop(0, CHUNK, zrow, ())

    row0 = s * ROWS_PER_TILE
    for k in range(ROWS_PER_TILE // CHUNK):
        pltpu.sync_copy(rows0, acc.at[pl.ds(row0 + k * CHUNK, CHUNK)])
    rem = ROWS_PER_TILE % CHUNK
    if rem:
        base = row0 + (ROWS_PER_TILE // CHUNK) * CHUNK
        pltpu.sync_copy(rows0.at[pl.ds(0, rem)], acc.at[pl.ds(base, rem)])

    plsc.subcore_barrier()

    nk = jnp.where(c == 0, K0, K1)
    cbase = jnp.where(c == 0, s * K0, NS * K0 + s * K1)
    ebase = cbase * CHUNK

    def sidx_load(b, g):
        pltpu.async_copy(src_hbm.at[pl.ds(ebase + g * CHUNK, CHUNK)],
                         sidx[b], sisem[b])

    def sidx_wait(b):
        pltpu.make_async_copy(src_hbm.at[pl.ds(0, CHUNK)], sidx[b],
                              sisem[b]).wait()

    def didx_load(b, g):
        pltpu.async_copy(dst_hbm.at[pl.ds(ebase + g * CHUNK, CHUNK)],
                         didx[b], disem[b])

    def didx_wait(b):
        pltpu.make_async_copy(dst_hbm.at[pl.ds(0, CHUNK)], didx[b],
                              disem[b]).wait()

    def gather_start(b):
        pltpu.async_copy(feat_hbm.at[sidx[b]], rows[b], gsem[b])

    def gather_wait(b):
        pltpu.make_async_copy(feat_hbm.at[sidx[b]], rows[b], gsem[b]).wait()

    def scatter_start(b):
        pltpu.async_copy(rows[b], acc.at[didx[b]], ssem[b], add=True)

    def scatter_wait(b):
        pltpu.make_async_copy(rows[b], acc.at[didx[b]], ssem[b]).wait()

    def body(b, g, first=False, no_next=False, no_sidx=False):
        ob = 1 - b
        gather_wait(b)
        if not first:
            scatter_wait(ob)
        if not no_next:
            sidx_wait(ob)
            gather_start(ob)
            if not no_sidx:
                sidx_load(b, g + 2)
            didx_load(ob, g + 1)
        didx_wait(b)
        scatter_start(b)

    sidx_load(0, 0)
    didx_load(0, 0)
    sidx_load(1, 1)
    sidx_wait(0)
    gather_start(0)

    body(0, 0, first=True)
    body(1, 1)

    def pair(p, _):
        g = 2 * p
        body(0, g)
        body(1, g + 1)
        return ()

    lax.fori_loop(1, (nk - 2) // 2, pair, ())

    body(0, nk - 2, no_sidx=True)
    body(1, nk - 1, no_next=True)
    scatter_wait(1)

    plsc.subcore_barrier()

    pltpu.sync_copy(acc.at[pl.ds(row0, ROWS_PER_TILE)],
                    out_hbm.at[c, pl.ds(row0, ROWS_PER_TILE)])


@jax.jit
def _sc_partials(feat_aug, src_p, dst_p):
    mesh = plsc.VectorSubcoreMesh(core_axis_name="c", subcore_axis_name="s")
    return pl.kernel(
        _sc_body,
        out_type=jax.ShapeDtypeStruct((NC, N_ACC, W_AUG), jnp.float32),
        mesh=mesh,
        scratch_types=[
            pltpu.VMEM((CHUNK, W_AUG), jnp.float32),
            pltpu.VMEM((CHUNK, W_AUG), jnp.float32),
            pltpu.VMEM((CHUNK,), jnp.int32),
            pltpu.VMEM((CHUNK,), jnp.int32),
            pltpu.VMEM((CHUNK,), jnp.int32),
            pltpu.VMEM((CHUNK,), jnp.int32),
            pltpu.VMEM_SHARED((N_ACC, W_AUG), jnp.float32),
        ] + [pltpu.SemaphoreType.DMA] * 8,
        compiler_params=pltpu.CompilerParams(use_tc_tiling_on_sc=False),
    )(feat_aug, src_p, dst_p)


RB = 400


def _tc_body(p_ref, wgc_ref, bgc_ref, wlt_ref, bl_ref, o_ref):
    x = p_ref[...]
    st = x[0] + x[1]
    deg = jnp.clip(st[:, D], 1.0, None)
    h = st[:, :D] / deg[:, None]
    ge = jnp.dot(h, wgc_ref[...], preferred_element_type=jnp.float32)
    ge = jnp.maximum(ge + bgc_ref[...], 0.0)
    top = jnp.max(ge, axis=1, keepdims=True)
    cnt = jnp.sum((ge == top).astype(jnp.float32), axis=1, keepdims=True)
    h2 = h * cnt
    o_ref[...] = (jnp.dot(h2, wlt_ref[...], preferred_element_type=jnp.float32)
                  + bl_ref[...])


@jax.jit
def _tc_finish(parts, W_gc, b_gc, W_lin_t, b_lin2d):
    grid = N_NODES // RB
    return pl.pallas_call(
        _tc_body,
        grid=(grid,),
        in_specs=[
            pl.BlockSpec((NC, RB, W_AUG), lambda i: (0, i, 0)),
            pl.BlockSpec((D, 3), lambda i: (0, 0)),
            pl.BlockSpec((1, 3), lambda i: (0, 0)),
            pl.BlockSpec((D, D), lambda i: (0, 0)),
            pl.BlockSpec((1, D), lambda i: (0, 0)),
        ],
        out_specs=pl.BlockSpec((RB, D), lambda i: (i, 0)),
        out_shape=jax.ShapeDtypeStruct((N_NODES, D), jnp.float32),
    )(parts, W_gc, b_gc, W_lin_t, b_lin2d)


def kernel(feature, edge_index, W_gc, b_gc, W_lin, b_lin):
    src = edge_index[0].astype(jnp.int32)
    dst = edge_index[1].astype(jnp.int32)
    e = src.shape[0]
    pad = E_PAD - e
    pad_src = jnp.arange(pad, dtype=jnp.int32) % N_NODES
    src_p = jnp.concatenate([src, pad_src])
    pad_dst = N_NODES + (jnp.arange(pad, dtype=jnp.int32) % (N_ACC - N_NODES))
    dst_p = jnp.concatenate([dst, pad_dst])
    feat_aug = jnp.concatenate(
        [feature,
         jnp.ones((N_NODES, 1), jnp.float32),
         jnp.zeros((N_NODES, W_AUG - D - 1), jnp.float32)], axis=1)
    parts = _sc_partials(feat_aug, src_p, dst_p)
    return _tc_finish(parts, W_gc, b_gc, W_lin.T, b_lin.reshape(1, D))


# speedup vs baseline: 3.5240x; 1.2322x over previous
import jax
import jax.numpy as jnp
from jax import lax
from jax.experimental import pallas as pl
from jax.experimental.pallas import tpu as pltpu
from jax.experimental.pallas import tpu_sc as plsc

N_NODES = 10000
D = 128

NC = 2
NS = 16
NW = NC * NS

CHUNK = 128
K0 = 80
K1 = 80
E_PAD = NS * (K0 + K1) * CHUNK
N_ACC = 10240
ROWS_PER_TILE = N_ACC // NS


def _sc_body(feat_hbm, src_hbm, dst_hbm, out_hbm, deg_hbm,
             rows0, rows1, sidx0, sidx1, didx0, didx1, deghist, acc,
             gsem0, gsem1, ssem0, ssem1, sisem0, sisem1, disem0, disem1):
    c = lax.axis_index("c")
    s = lax.axis_index("s")
    wid = s * NC + c

    rows = (rows0, rows1)
    sidx = (sidx0, sidx1)
    didx = (didx0, didx1)
    gsem = (gsem0, gsem1)
    ssem = (ssem0, ssem1)
    sisem = (sisem0, sisem1)
    disem = (disem0, disem1)

    zero16 = jnp.zeros((16,), jnp.float32)

    def zrow(i, _):
        def zcol(j, _):
            rows0[i, pl.ds(j * 16, 16)] = zero16
            return ()
        return lax.fori_loop(0, D // 16, zcol, ())

    lax.fori_loop(0, CHUNK, zrow, ())

    def zdeg(i, _):
        deghist[pl.ds(i * 16, 16)] = zero16
        return ()

    lax.fori_loop(0, N_ACC // 16, zdeg, ())

    row0 = s * ROWS_PER_TILE
    for k in range(ROWS_PER_TILE // CHUNK):
        pltpu.sync_copy(rows0, acc.at[pl.ds(row0 + k * CHUNK, CHUNK)])

    plsc.subcore_barrier()

    nk = jnp.where(c == 0, K0, K1)
    cbase = jnp.where(c == 0, s * K0, NS * K0 + s * K1)
    ebase = cbase * CHUNK

    def sidx_load(b, g):
        pltpu.async_copy(src_hbm.at[pl.ds(ebase + g * CHUNK, CHUNK)],
                         sidx[b], sisem[b])

    def sidx_wait(b):
        pltpu.make_async_copy(src_hbm.at[pl.ds(0, CHUNK)], sidx[b],
                              sisem[b]).wait()

    def didx_load(b, g):
        pltpu.async_copy(dst_hbm.at[pl.ds(ebase + g * CHUNK, CHUNK)],
                         didx[b], disem[b])

    def didx_wait(b):
        pltpu.make_async_copy(dst_hbm.at[pl.ds(0, CHUNK)], didx[b],
                              disem[b]).wait()

    def gather_start(b):
        pltpu.async_copy(feat_hbm.at[sidx[b]], rows[b], gsem[b])

    def gather_wait(b):
        pltpu.make_async_copy(feat_hbm.at[sidx[b]], rows[b], gsem[b]).wait()

    def scatter_start(b):
        pltpu.async_copy(rows[b], acc.at[didx[b]], ssem[b], add=True)

    def scatter_wait(b):
        pltpu.make_async_copy(rows[b], acc.at[didx[b]], ssem[b]).wait()

    ones16 = jnp.ones((16,), jnp.float32)

    def hist_update(b):
        def hstep(j, _):
            idx16 = didx[b][pl.ds(j * 16, 16)]
            plsc.addupdate_scatter(deghist, (idx16,), ones16)
            return ()
        lax.fori_loop(0, CHUNK // 16, hstep, ())

    def body(b, g, first=False, no_next=False, no_sidx=False):
        ob = 1 - b
        gather_wait(b)
        if not first:
            scatter_wait(ob)
        if not no_next:
            sidx_wait(ob)
            gather_start(ob)
            if not no_sidx:
                sidx_load(b, g + 2)
            didx_load(ob, g + 1)
        didx_wait(b)
        scatter_start(b)
        hist_update(b)

    sidx_load(0, 0)
    didx_load(0, 0)
    sidx_load(1, 1)
    sidx_wait(0)
    gather_start(0)

    body(0, 0, first=True)
    body(1, 1)

    def pair(p, _):
        g = 2 * p
        body(0, g)
        body(1, g + 1)
        return ()

    lax.fori_loop(1, (nk - 2) // 2, pair, ())

    body(0, nk - 2, no_sidx=True)
    body(1, nk - 1, no_next=True)
    scatter_wait(1)

    plsc.subcore_barrier()

    pltpu.sync_copy(acc.at[pl.ds(row0, ROWS_PER_TILE)],
                    out_hbm.at[c, pl.ds(row0, ROWS_PER_TILE)])
    pltpu.sync_copy(deghist, deg_hbm.at[wid])


@jax.jit
def _sc_partials(feature, src_p, dst_p):
    mesh = plsc.VectorSubcoreMesh(core_axis_name="c", subcore_axis_name="s")
    return pl.kernel(
        _sc_body,
        out_type=[
            jax.ShapeDtypeStruct((NC, N_ACC, D), jnp.float32),
            jax.ShapeDtypeStruct((NW, N_ACC), jnp.float32),
        ],
        mesh=mesh,
        scratch_types=[
            pltpu.VMEM((CHUNK, D), jnp.float32),
            pltpu.VMEM((CHUNK, D), jnp.float32),
            pltpu.VMEM((CHUNK,), jnp.int32),
            pltpu.VMEM((CHUNK,), jnp.int32),
            pltpu.VMEM((CHUNK,), jnp.int32),
            pltpu.VMEM((CHUNK,), jnp.int32),
            pltpu.VMEM((N_ACC,), jnp.float32),
            pltpu.VMEM_SHARED((N_ACC, D), jnp.float32),
        ] + [pltpu.SemaphoreType.DMA] * 8,
        compiler_params=pltpu.CompilerParams(needs_layout_passes=False),
    )(feature, src_p, dst_p)


RB = 512


def _tc_body(p_ref, dp_ref, wgc_ref, bgc_ref, wlt_ref, bl_ref, o_ref):
    x = p_ref[...]
    st = x[0] + x[1]
    deg = jnp.sum(dp_ref[...], axis=0)
    h = st / jnp.clip(deg, 1.0, None)[:, None]
    ge = jnp.dot(h, wgc_ref[...], preferred_element_type=jnp.float32)
    ge = jnp.maximum(ge + bgc_ref[...], 0.0)
    top = jnp.max(ge, axis=1, keepdims=True)
    cnt = jnp.sum((ge == top).astype(jnp.float32), axis=1, keepdims=True)
    h2 = h * cnt
    o_ref[...] = (jnp.dot(h2, wlt_ref[...], preferred_element_type=jnp.float32)
                  + bl_ref[...])


@jax.jit
def _tc_finish(parts, degp, W_gc, b_gc, W_lin_t, b_lin2d):
    grid = N_ACC // RB
    return pl.pallas_call(
        _tc_body,
        grid=(grid,),
        in_specs=[
            pl.BlockSpec((NC, RB, D), lambda i: (0, i, 0)),
            pl.BlockSpec((NW, RB), lambda i: (0, i)),
            pl.BlockSpec((D, 3), lambda i: (0, 0)),
            pl.BlockSpec((1, 3), lambda i: (0, 0)),
            pl.BlockSpec((D, D), lambda i: (0, 0)),
            pl.BlockSpec((1, D), lambda i: (0, 0)),
        ],
        out_specs=pl.BlockSpec((RB, D), lambda i: (i, 0)),
        out_shape=jax.ShapeDtypeStruct((N_ACC, D), jnp.float32),
    )(parts, degp, W_gc, b_gc, W_lin_t, b_lin2d)


def kernel(feature, edge_index, W_gc, b_gc, W_lin, b_lin):
    src = edge_index[0].astype(jnp.int32)
    dst = edge_index[1].astype(jnp.int32)
    e = src.shape[0]
    pad = E_PAD - e
    pad_src = jnp.arange(pad, dtype=jnp.int32) % N_NODES
    pad_dst = N_NODES + (jnp.arange(pad, dtype=jnp.int32) % (N_ACC - N_NODES))
    src_p = jnp.concatenate([src, pad_src])
    dst_p = jnp.concatenate([dst, pad_dst])
    parts, degp = _sc_partials(feature, src_p, dst_p)
    out = _tc_finish(parts, degp, W_gc, b_gc, W_lin.T, b_lin.reshape(1, D))
    return out[:N_NODES]
